# baseline (speedup 1.0000x reference)
import functools

import jax
import jax.numpy as jnp
from jax.experimental import pallas as pl
from jax.experimental.pallas import tpu as pltpu

_ONES_W = 128


def _p1_kernel(a_ref, x_ref, m1_ref, invn_ref, s_scr, *, nsteps, d):
    i = pl.program_id(0)
    a = a_ref[...]
    y = x_ref[...]
    y2 = (y * y).astype(jnp.bfloat16)
    ones_rows = jnp.ones((y.shape[0], _ONES_W), jnp.bfloat16)
    y_aug = jnp.concatenate([y2, ones_rows], axis=1)
    part = jax.lax.dot_general(
        y_aug, a, (((0,), (0,)), ((), ())), preferred_element_type=jnp.float32
    )

    @pl.when(i == 0)
    def _init():
        s_scr[...] = part

    @pl.when(i > 0)
    def _acc():
        s_scr[...] += part

    @pl.when(i == nsteps - 1)
    def _finish():
        invn = 1.0 / jnp.maximum(s_scr[d : d + 1, :], 1.0)
        m1_ref[...] = jnp.transpose((s_scr[:d, :] * invn).astype(jnp.bfloat16))
        invn_ref[...] = invn


def _p2_kernel(a_ref, m1_ref, w1_ref, invn_ref, m2_ref, inve_ref, s_scr,
               *, nsteps, d):
    i = pl.program_id(0)
    a = a_ref[...]
    ones_cols = jnp.ones((a.shape[1], _ONES_W), jnp.bfloat16)
    m1_aug = jnp.concatenate([m1_ref[...], ones_cols], axis=1)
    z_aug = jnp.dot(a, m1_aug, preferred_element_type=jnp.float32)
    inve = 1.0 / jnp.maximum(z_aug[:, d : d + 1], 1.0)
    inve_ref[...] = inve
    u = jnp.sqrt(z_aug[:, :d] * inve)
    h = jnp.maximum(
        jnp.dot(u.astype(jnp.bfloat16), w1_ref[...],
                preferred_element_type=jnp.float32),
        0.0,
    )
    h2 = (h * h).astype(jnp.bfloat16)
    part = jax.lax.dot_general(
        h2, a, (((0,), (0,)), ((), ())), preferred_element_type=jnp.float32
    )

    @pl.when(i == 0)
    def _init():
        s_scr[...] = part

    @pl.when(i > 0)
    def _acc():
        s_scr[...] += part

    @pl.when(i == nsteps - 1)
    def _finish():
        m2_ref[...] = jnp.transpose(
            (s_scr[...] * invn_ref[...]).astype(jnp.bfloat16)
        )


def _p3_kernel(a_ref, m2_ref, w2_ref, inve_ref, out_ref):
    a = a_ref[...]
    z = jnp.dot(a, m2_ref[...], preferred_element_type=jnp.float32)
    u = jnp.sqrt(z * inve_ref[...])
    out_ref[...] = jnp.maximum(
        jnp.dot(u.astype(jnp.bfloat16), w2_ref[...],
                preferred_element_type=jnp.float32),
        0.0,
    )


def kernel(x_0, incidence, W1, W2):
    n, d = x_0.shape
    e = incidence.shape[1]
    ep = 2048
    bn = 2000
    nsteps = n // bn

    a16 = jnp.concatenate(
        [incidence.astype(jnp.bfloat16), jnp.zeros((n, ep - e), jnp.bfloat16)],
        axis=1,
    )
    w1_16 = W1.astype(jnp.bfloat16)
    w2_16 = W2.astype(jnp.bfloat16)

    m1, invn = pl.pallas_call(
        functools.partial(_p1_kernel, nsteps=nsteps, d=d),
        grid=(nsteps,),
        in_specs=[
            pl.BlockSpec((bn, ep), lambda i: (i, 0)),
            pl.BlockSpec((bn, d), lambda i: (i, 0)),
        ],
        out_specs=[
            pl.BlockSpec((ep, d), lambda i: (0, 0)),
            pl.BlockSpec((1, ep), lambda i: (0, 0)),
        ],
        out_shape=[
            jax.ShapeDtypeStruct((ep, d), jnp.bfloat16),
            jax.ShapeDtypeStruct((1, ep), jnp.float32),
        ],
        scratch_shapes=[pltpu.VMEM((d + _ONES_W, ep), jnp.float32)],
    )(a16, x_0)

    m2, inve = pl.pallas_call(
        functools.partial(_p2_kernel, nsteps=nsteps, d=d),
        grid=(nsteps,),
        in_specs=[
            pl.BlockSpec((bn, ep), lambda i: (i, 0)),
            pl.BlockSpec((ep, d), lambda i: (0, 0)),
            pl.BlockSpec((d, d), lambda i: (0, 0)),
            pl.BlockSpec((1, ep), lambda i: (0, 0)),
        ],
        out_specs=[
            pl.BlockSpec((ep, d), lambda i: (0, 0)),
            pl.BlockSpec((bn, 1), lambda i: (i, 0)),
        ],
        out_shape=[
            jax.ShapeDtypeStruct((ep, d), jnp.bfloat16),
            jax.ShapeDtypeStruct((n, 1), jnp.float32),
        ],
        scratch_shapes=[pltpu.VMEM((d, ep), jnp.float32)],
    )(a16, m1, w1_16, invn)

    out = pl.pallas_call(
        _p3_kernel,
        grid=(nsteps,),
        in_specs=[
            pl.BlockSpec((bn, ep), lambda i: (i, 0)),
            pl.BlockSpec((ep, d), lambda i: (0, 0)),
            pl.BlockSpec((d, d), lambda i: (0, 0)),
            pl.BlockSpec((bn, 1), lambda i: (i, 0)),
        ],
        out_specs=pl.BlockSpec((bn, d), lambda i: (i, 0)),
        out_shape=jax.ShapeDtypeStruct((n, d), jnp.float32),
    )(a16, m2, w2_16, inve)

    return out


# speedup vs baseline: 1.9545x; 1.9545x over previous
import functools

import jax
import jax.numpy as jnp
from jax.experimental import pallas as pl
from jax.experimental.pallas import tpu as pltpu

_ONES_W = 128


def _p1_kernel(a_ref, x_ref, m1_ref, invn_ref, s_scr, *, nsteps, d):
    i = pl.program_id(0)
    a = a_ref[...]
    y = x_ref[...]
    y2 = (y * y).astype(jnp.bfloat16)
    ones_rows = jnp.ones((y.shape[0], _ONES_W), jnp.bfloat16)
    y_aug = jnp.concatenate([y2, ones_rows], axis=1)
    part = jax.lax.dot_general(
        y_aug, a, (((0,), (0,)), ((), ())), preferred_element_type=jnp.float32
    )

    @pl.when(i == 0)
    def _init():
        s_scr[...] = part

    @pl.when(i > 0)
    def _acc():
        s_scr[...] += part

    @pl.when(i == nsteps - 1)
    def _finish():
        invn = 1.0 / jnp.maximum(s_scr[d : d + 1, :], 1.0)
        m1_ref[...] = jnp.transpose((s_scr[:d, :] * invn).astype(jnp.bfloat16))
        invn_ref[...] = invn


def _p2_kernel(a_ref, m1_ref, w1_ref, invn_ref, m2_ref, inve_ref, s_scr,
               *, nsteps, d):
    i = pl.program_id(0)
    a = a_ref[...]
    ones_cols = jnp.ones((a.shape[1], _ONES_W), jnp.bfloat16)
    m1_aug = jnp.concatenate([m1_ref[...], ones_cols], axis=1)
    z_aug = jnp.dot(a, m1_aug, preferred_element_type=jnp.float32)
    inve = 1.0 / jnp.maximum(z_aug[:, d : d + 1], 1.0)
    inve_ref[...] = inve
    u = jnp.sqrt(z_aug[:, :d] * inve)
    h = jnp.maximum(
        jnp.dot(u.astype(jnp.bfloat16), w1_ref[...],
                preferred_element_type=jnp.float32),
        0.0,
    )
    h2 = (h * h).astype(jnp.bfloat16)
    part = jax.lax.dot_general(
        h2, a, (((0,), (0,)), ((), ())), preferred_element_type=jnp.float32
    )

    @pl.when(i == 0)
    def _init():
        s_scr[...] = part

    @pl.when(i > 0)
    def _acc():
        s_scr[...] += part

    @pl.when(i == nsteps - 1)
    def _finish():
        m2_ref[...] = jnp.transpose(
            (s_scr[...] * invn_ref[...]).astype(jnp.bfloat16)
        )


def _p3_kernel(a_ref, m2_ref, w2_ref, inve_ref, out_ref):
    a = a_ref[...]
    z = jnp.dot(a, m2_ref[...], preferred_element_type=jnp.float32)
    u = jnp.sqrt(z * inve_ref[...])
    out_ref[...] = jnp.maximum(
        jnp.dot(u.astype(jnp.bfloat16), w2_ref[...],
                preferred_element_type=jnp.float32),
        0.0,
    )


def kernel(x_0, incidence, W1, W2):
    n, d = x_0.shape
    e = incidence.shape[1]
    bn = 2000
    nsteps = n // bn

    a16 = incidence.astype(jnp.bfloat16)
    w1_16 = W1.astype(jnp.bfloat16)
    w2_16 = W2.astype(jnp.bfloat16)

    m1, invn = pl.pallas_call(
        functools.partial(_p1_kernel, nsteps=nsteps, d=d),
        grid=(nsteps,),
        in_specs=[
            pl.BlockSpec((bn, e), lambda i: (i, 0)),
            pl.BlockSpec((bn, d), lambda i: (i, 0)),
        ],
        out_specs=[
            pl.BlockSpec((e, d), lambda i: (0, 0)),
            pl.BlockSpec((1, e), lambda i: (0, 0)),
        ],
        out_shape=[
            jax.ShapeDtypeStruct((e, d), jnp.bfloat16),
            jax.ShapeDtypeStruct((1, e), jnp.float32),
        ],
        scratch_shapes=[pltpu.VMEM((d + _ONES_W, e), jnp.float32)],
    )(a16, x_0)

    m2, inve = pl.pallas_call(
        functools.partial(_p2_kernel, nsteps=nsteps, d=d),
        grid=(nsteps,),
        in_specs=[
            pl.BlockSpec((bn, e), lambda i: (i, 0)),
            pl.BlockSpec((e, d), lambda i: (0, 0)),
            pl.BlockSpec((d, d), lambda i: (0, 0)),
            pl.BlockSpec((1, e), lambda i: (0, 0)),
        ],
        out_specs=[
            pl.BlockSpec((e, d), lambda i: (0, 0)),
            pl.BlockSpec((bn, 1), lambda i: (i, 0)),
        ],
        out_shape=[
            jax.ShapeDtypeStruct((e, d), jnp.bfloat16),
            jax.ShapeDtypeStruct((n, 1), jnp.float32),
        ],
        scratch_shapes=[pltpu.VMEM((d, e), jnp.float32)],
    )(a16, m1, w1_16, invn)

    out = pl.pallas_call(
        _p3_kernel,
        grid=(nsteps,),
        in_specs=[
            pl.BlockSpec((bn, e), lambda i: (i, 0)),
            pl.BlockSpec((e, d), lambda i: (0, 0)),
            pl.BlockSpec((d, d), lambda i: (0, 0)),
            pl.BlockSpec((bn, 1), lambda i: (i, 0)),
        ],
        out_specs=pl.BlockSpec((bn, d), lambda i: (i, 0)),
        out_shape=jax.ShapeDtypeStruct((n, d), jnp.float32),
    )(a16, m2, w2_16, inve)

    return out
